# baseline (speedup 1.0000x reference)
import jax
import jax.numpy as jnp
from jax.experimental import pallas as pl
from jax.experimental.pallas import tpu as pltpu

QUEUE_ROWS = 32768


def _fifo_copy(x_ref, q_ref, out_ref, sem_q, sem_x):
    shift = x_ref.shape[0]
    keep = QUEUE_ROWS - shift
    cq = pltpu.make_async_copy(
        q_ref.at[pl.ds(shift, keep)], out_ref.at[pl.ds(0, keep)], sem_q
    )
    cx = pltpu.make_async_copy(x_ref, out_ref.at[pl.ds(keep, shift)], sem_x)
    cq.start()
    cx.start()
    cq.wait()
    cx.wait()


def kernel(x, queue):
    return pl.pallas_call(
        _fifo_copy,
        out_shape=jax.ShapeDtypeStruct(queue.shape, queue.dtype),
        in_specs=[
            pl.BlockSpec(memory_space=pl.ANY),
            pl.BlockSpec(memory_space=pl.ANY),
        ],
        out_specs=pl.BlockSpec(memory_space=pl.ANY),
        scratch_shapes=[pltpu.SemaphoreType.DMA, pltpu.SemaphoreType.DMA],
    )(x, queue)


# speedup vs baseline: 1.0097x; 1.0097x over previous
import jax
import jax.numpy as jnp
from jax.experimental import pallas as pl
from jax.experimental.pallas import tpu as pltpu

QUEUE_ROWS = 32768


N_Q_CHUNKS = 32
N_X_CHUNKS = 8


def _fifo_copy(x_ref, q_ref, out_ref, sem_q, sem_x):
    shift = x_ref.shape[0]
    keep = QUEUE_ROWS - shift
    qc = keep // N_Q_CHUNKS
    xc = shift // N_X_CHUNKS
    copies = []
    for i in range(N_Q_CHUNKS):
        copies.append(pltpu.make_async_copy(
            q_ref.at[pl.ds(shift + i * qc, qc)],
            out_ref.at[pl.ds(i * qc, qc)], sem_q))
    for i in range(N_X_CHUNKS):
        copies.append(pltpu.make_async_copy(
            x_ref.at[pl.ds(i * xc, xc)],
            out_ref.at[pl.ds(keep + i * xc, xc)], sem_x))
    for c in copies:
        c.start()
    for c in copies:
        c.wait()


def kernel(x, queue):
    return pl.pallas_call(
        _fifo_copy,
        out_shape=jax.ShapeDtypeStruct(queue.shape, queue.dtype),
        in_specs=[
            pl.BlockSpec(memory_space=pl.ANY),
            pl.BlockSpec(memory_space=pl.ANY),
        ],
        out_specs=pl.BlockSpec(memory_space=pl.ANY),
        scratch_shapes=[pltpu.SemaphoreType.DMA, pltpu.SemaphoreType.DMA],
    )(x, queue)


# speedup vs baseline: 28.4539x; 28.1797x over previous
import jax
import jax.numpy as jnp
from jax.experimental import pallas as pl
from jax.experimental.pallas import tpu as pltpu

QUEUE_ROWS = 32768
BLOCK = 2048


def _fifo_copy(q_ref, x_ref, o_ref, *, n_q_blocks):
    i = pl.program_id(0)

    @pl.when(i < n_q_blocks)
    def _():
        o_ref[...] = q_ref[...]

    @pl.when(i >= n_q_blocks)
    def _():
        o_ref[...] = x_ref[...]


def kernel(x, queue):
    import functools

    shift = x.shape[0]
    assert shift % BLOCK == 0 and QUEUE_ROWS % BLOCK == 0
    n_blocks = QUEUE_ROWS // BLOCK
    n_x_blocks = shift // BLOCK
    n_q_blocks = n_blocks - n_x_blocks
    shift_blocks = shift // BLOCK

    return pl.pallas_call(
        functools.partial(_fifo_copy, n_q_blocks=n_q_blocks),
        grid=(n_blocks,),
        in_specs=[
            pl.BlockSpec(
                (BLOCK, queue.shape[1]),
                lambda i: (jnp.minimum(i + shift_blocks, n_blocks - 1), 0),
            ),
            pl.BlockSpec(
                (BLOCK, x.shape[1]),
                lambda i: (jnp.clip(i - n_q_blocks, 0, n_x_blocks - 1), 0),
            ),
        ],
        out_specs=pl.BlockSpec((BLOCK, queue.shape[1]), lambda i: (i, 0)),
        out_shape=jax.ShapeDtypeStruct(queue.shape, queue.dtype),
        compiler_params=pltpu.CompilerParams(
            dimension_semantics=("arbitrary",),
        ),
    )(queue, x)


# speedup vs baseline: 38.5350x; 1.3543x over previous
import jax
import jax.numpy as jnp
from jax.experimental import pallas as pl
from jax.experimental.pallas import tpu as pltpu

QUEUE_ROWS = 32768
BLOCK = 4096


def _fifo_copy(q_ref, x_ref, o_ref, *, n_q_blocks):
    i = pl.program_id(0)

    @pl.when(i < n_q_blocks)
    def _():
        o_ref[...] = q_ref[...]

    @pl.when(i >= n_q_blocks)
    def _():
        o_ref[...] = x_ref[...]


def kernel(x, queue):
    import functools

    shift = x.shape[0]
    assert shift % BLOCK == 0 and QUEUE_ROWS % BLOCK == 0
    n_blocks = QUEUE_ROWS // BLOCK
    n_x_blocks = shift // BLOCK
    n_q_blocks = n_blocks - n_x_blocks
    shift_blocks = shift // BLOCK

    return pl.pallas_call(
        functools.partial(_fifo_copy, n_q_blocks=n_q_blocks),
        grid=(n_blocks,),
        in_specs=[
            pl.BlockSpec(
                (BLOCK, queue.shape[1]),
                lambda i: (jnp.minimum(i + shift_blocks, n_blocks - 1), 0),
            ),
            pl.BlockSpec(
                (BLOCK, x.shape[1]),
                lambda i: (jnp.clip(i - n_q_blocks, 0, n_x_blocks - 1), 0),
            ),
        ],
        out_specs=pl.BlockSpec((BLOCK, queue.shape[1]), lambda i: (i, 0)),
        out_shape=jax.ShapeDtypeStruct(queue.shape, queue.dtype),
        compiler_params=pltpu.CompilerParams(
            dimension_semantics=("arbitrary",),
        ),
    )(queue, x)
